# Initial kernel scaffold; baseline (speedup 1.0000x reference)
import jax
import jax.numpy as jnp
from jax.experimental import pallas as pl


def kernel(contexts, t, in_emb, out_emb):
    raise NotImplementedError("write your pallas kernel here")


# speedup vs baseline: 7.2154x; 7.2154x over previous
import functools
import jax
import jax.numpy as jnp
from jax import lax
from jax.experimental import pallas as pl
from jax.experimental.pallas import tpu as pltpu
from jax.experimental.pallas import tpu_sc as plsc

VOCAB = 100000
H = 64
BATCH = 4096
C = 50
T = 50
NC = 2
NS = 16
NW = NC * NS
BPW = BATCH // NW
NQ = H // 16

_mesh = plsc.VectorSubcoreMesh(core_axis_name="c", subcore_axis_name="s")


TP = 64


@functools.partial(
    pl.kernel,
    mesh=_mesh,
    compiler_params=pltpu.CompilerParams(use_tc_tiling_on_sc=False),
    out_type=jax.ShapeDtypeStruct((BATCH, TP), jnp.float32),
    scratch_types=[
        pltpu.VMEM((BPW, C), jnp.int32),
        pltpu.VMEM((BPW, T), jnp.int32),
        pltpu.VMEM((C, H), jnp.float32),
        pltpu.VMEM((T, H), jnp.float32),
        pltpu.VMEM((BPW, TP), jnp.float32),
        pltpu.SemaphoreType.DMA,
        pltpu.SemaphoreType.DMA,
    ],
)
def _cbow_sc(ctx_hbm, t_hbm, in_hbm, out_hbm, y_hbm,
             ctx_idx, t_idx, crows, trows, yb, sem_c, sem_t):
    wid = lax.axis_index("s") * NC + lax.axis_index("c")
    base = wid * BPW
    pltpu.sync_copy(ctx_hbm.at[pl.ds(base, BPW)], ctx_idx)
    pltpu.sync_copy(t_hbm.at[pl.ds(base, BPW)], t_idx)
    lane = lax.iota(jnp.int32, 16)
    perms = [(lane + (1 << k)) % 16 for k in range(4)]

    def hsum(v):
        for pidx in perms:
            v = v + jnp.take(v, pidx, mode="wrap")
        return v

    def body(b, carry):
        cp_c = pltpu.async_copy(in_hbm.at[ctx_idx.at[b]], crows, sem_c)
        cp_t = pltpu.async_copy(out_hbm.at[t_idx.at[b]], trows, sem_t)
        cp_c.wait()
        acc = [jnp.zeros((16,), jnp.float32) for _ in range(NQ)]
        for c in range(C):
            for q in range(NQ):
                acc[q] = acc[q] + crows[c, pl.ds(q * 16, 16)]
        cp_t.wait()
        for g in range(TP // 16):
            yv = jnp.zeros((16,), jnp.float32)
            for jj in range(16):
                j = g * 16 + jj
                if j >= T:
                    continue
                p = trows[j, pl.ds(0, 16)] * acc[0]
                for q in range(1, NQ):
                    p = p + trows[j, pl.ds(q * 16, 16)] * acc[q]
                yv = jnp.where(lane == jj, hsum(p), yv)
            yb[b, pl.ds(g * 16, 16)] = yv
        return carry

    lax.fori_loop(0, BPW, body, 0)
    pltpu.sync_copy(yb, y_hbm.at[pl.ds(base, BPW)])


def kernel(contexts, t, in_emb, out_emb):
    y = _cbow_sc(contexts.astype(jnp.int32), t.astype(jnp.int32),
                 in_emb, out_emb)
    return y[:, :T].reshape(BATCH, 1, T)


# speedup vs baseline: 8.0184x; 1.1113x over previous
import functools
import jax
import jax.numpy as jnp
from jax import lax
from jax.experimental import pallas as pl
from jax.experimental.pallas import tpu as pltpu
from jax.experimental.pallas import tpu_sc as plsc

VOCAB = 100000
H = 64
BATCH = 4096
C = 50
T = 50
NC = 2
NS = 16
NW = NC * NS
BPW = BATCH // NW
NQ = H // 16
TP = 64
NB = 2
NCHUNK = BPW // NB
CH = NB * C

_mesh = plsc.VectorSubcoreMesh(core_axis_name="c", subcore_axis_name="s")


@functools.partial(
    pl.kernel,
    mesh=_mesh,
    compiler_params=pltpu.CompilerParams(use_tc_tiling_on_sc=False),
    out_type=jax.ShapeDtypeStruct((BATCH, TP), jnp.float32),
    scratch_types=[
        pltpu.VMEM((NCHUNK, CH), jnp.int32),
        pltpu.VMEM((NCHUNK, CH), jnp.int32),
        pltpu.VMEM((CH, H), jnp.float32),
        pltpu.VMEM((CH, H), jnp.float32),
        pltpu.VMEM((CH, H), jnp.float32),
        pltpu.VMEM((CH, H), jnp.float32),
        pltpu.VMEM((BPW, TP), jnp.float32),
        pltpu.SemaphoreType.DMA,
        pltpu.SemaphoreType.DMA,
    ],
)
def _cbow_sc(ctx_hbm, t_hbm, in_hbm, out_hbm, y_hbm,
             ctx_idx, t_idx, crows0, crows1, trows0, trows1, yb,
             sem0, sem1):
    wid = lax.axis_index("s") * NC + lax.axis_index("c")
    base = wid * NCHUNK
    pltpu.sync_copy(ctx_hbm.at[pl.ds(base, NCHUNK)], ctx_idx)
    pltpu.sync_copy(t_hbm.at[pl.ds(base, NCHUNK)], t_idx)
    lane = lax.iota(jnp.int32, 16)
    perms = [(lane + (1 << k)) % 16 for k in range(4)]

    def hsum(v):
        for pidx in perms:
            v = v + jnp.take(v, pidx, mode="wrap")
        return v

    def fire(k, crows, trows, sem):
        pltpu.async_copy(in_hbm.at[ctx_idx.at[k]], crows, sem)
        pltpu.async_copy(out_hbm.at[t_idx.at[k]], trows, sem)

    def drain(crows, trows, sem):
        pltpu.make_async_copy(in_hbm.at[pl.ds(0, CH)], crows, sem).wait()
        pltpu.make_async_copy(in_hbm.at[pl.ds(0, CH)], trows, sem).wait()

    def compute(k, crows, trows):
        for bb in range(NB):
            acc = [jnp.zeros((16,), jnp.float32) for _ in range(2 * NQ)]
            for c in range(C):
                for q in range(NQ):
                    a = (c % 2) * NQ + q
                    acc[a] = acc[a] + crows[bb * C + c, pl.ds(q * 16, 16)]
            con = [acc[q] + acc[NQ + q] for q in range(NQ)]
            b = k * NB + bb
            for g in range(TP // 16):
                yv = jnp.zeros((16,), jnp.float32)
                for jj in range(16):
                    j = g * 16 + jj
                    if j >= T:
                        continue
                    p = trows[bb * C + j, pl.ds(0, 16)] * con[0]
                    for q in range(1, NQ):
                        p = p + trows[bb * C + j, pl.ds(q * 16, 16)] * con[q]
                    yv = jnp.where(lane == jj, hsum(p), yv)
                yb[b, pl.ds(g * 16, 16)] = yv

    fire(0, crows0, trows0, sem0)

    def body(i, carry):
        k0 = 2 * i
        fire(k0 + 1, crows1, trows1, sem1)
        drain(crows0, trows0, sem0)
        compute(k0, crows0, trows0)
        fire(jnp.minimum(k0 + 2, NCHUNK - 1), crows0, trows0, sem0)
        drain(crows1, trows1, sem1)
        compute(k0 + 1, crows1, trows1)
        return carry

    lax.fori_loop(0, NCHUNK // 2, body, 0)
    drain(crows0, trows0, sem0)
    pltpu.sync_copy(yb, y_hbm.at[pl.ds(wid * BPW, BPW)])


def kernel(contexts, t, in_emb, out_emb):
    ctx2 = contexts.astype(jnp.int32).reshape(NW * NCHUNK, CH)
    t2 = t.astype(jnp.int32).reshape(NW * NCHUNK, CH)
    y = _cbow_sc(ctx2, t2, in_emb, out_emb)
    return y[:, :T].reshape(BATCH, 1, T)


# speedup vs baseline: 9.2543x; 1.1541x over previous
import functools
import jax
import jax.numpy as jnp
from jax import lax
from jax.experimental import pallas as pl
from jax.experimental.pallas import tpu as pltpu
from jax.experimental.pallas import tpu_sc as plsc

VOCAB = 100000
H = 64
BATCH = 4096
C = 50
T = 50
NC = 2
NS = 16
NW = NC * NS
BPW = BATCH // NW
NQ = H // 16
TP = 64

_mesh = plsc.VectorSubcoreMesh(core_axis_name="c", subcore_axis_name="s")


@functools.partial(
    pl.kernel,
    mesh=_mesh,
    compiler_params=pltpu.CompilerParams(use_tc_tiling_on_sc=False),
    out_type=jax.ShapeDtypeStruct((BATCH, TP), jnp.float32),
    scratch_types=[
        pltpu.VMEM((BPW, C), jnp.int32),
        pltpu.VMEM((BPW, T), jnp.int32),
        pltpu.VMEM((C, H), jnp.float32),
        pltpu.VMEM((C, H), jnp.float32),
        pltpu.VMEM((T, H), jnp.float32),
        pltpu.VMEM((T, H), jnp.float32),
        pltpu.VMEM((BPW, TP), jnp.float32),
        pltpu.SemaphoreType.DMA,
        pltpu.SemaphoreType.DMA,
    ],
)
def _cbow_sc(ctx_hbm, t_hbm, in_hbm, out_hbm, y_hbm,
             ctx_idx, t_idx, crows0, crows1, trows0, trows1, yb,
             sem0, sem1):
    wid = lax.axis_index("s") * NC + lax.axis_index("c")
    base = wid * BPW
    pltpu.sync_copy(ctx_hbm.at[pl.ds(base, BPW)], ctx_idx)
    pltpu.sync_copy(t_hbm.at[pl.ds(base, BPW)], t_idx)
    lane = lax.iota(jnp.int32, 16)
    perms = [(lane + (1 << k)) % 16 for k in range(4)]

    def hsum(v):
        for pidx in perms:
            v = v + jnp.take(v, pidx, mode="wrap")
        return v

    def fire(b, crows, trows, sem):
        pltpu.async_copy(in_hbm.at[ctx_idx.at[b]], crows, sem)
        pltpu.async_copy(out_hbm.at[t_idx.at[b]], trows, sem)

    def drain(crows, trows, sem):
        pltpu.make_async_copy(in_hbm.at[pl.ds(0, C)], crows, sem).wait()
        pltpu.make_async_copy(in_hbm.at[pl.ds(0, T)], trows, sem).wait()

    def compute(b, crows, trows):
        acc = [jnp.zeros((16,), jnp.float32) for _ in range(2 * NQ)]
        for c in range(C):
            for q in range(NQ):
                a = (c % 2) * NQ + q
                acc[a] = acc[a] + crows[c, pl.ds(q * 16, 16)]
        con = [acc[q] + acc[NQ + q] for q in range(NQ)]
        for g in range(TP // 16):
            yv = jnp.zeros((16,), jnp.float32)
            for jj in range(16):
                j = g * 16 + jj
                if j >= T:
                    continue
                p = trows[j, pl.ds(0, 16)] * con[0]
                for q in range(1, NQ):
                    p = p + trows[j, pl.ds(q * 16, 16)] * con[q]
                yv = jnp.where(lane == jj, hsum(p), yv)
            yb[b, pl.ds(g * 16, 16)] = yv

    fire(0, crows0, trows0, sem0)

    def body(i, carry):
        b0 = 2 * i
        fire(b0 + 1, crows1, trows1, sem1)
        drain(crows0, trows0, sem0)
        compute(b0, crows0, trows0)
        fire(jnp.minimum(b0 + 2, BPW - 1), crows0, trows0, sem0)
        drain(crows1, trows1, sem1)
        compute(b0 + 1, crows1, trows1)
        return carry

    lax.fori_loop(0, BPW // 2, body, 0)
    drain(crows0, trows0, sem0)
    pltpu.sync_copy(yb, y_hbm.at[pl.ds(base, BPW)])


def kernel(contexts, t, in_emb, out_emb):
    y = _cbow_sc(contexts.astype(jnp.int32), t.astype(jnp.int32),
                 in_emb, out_emb)
    return y[:, :T].reshape(BATCH, 1, T)
